# baseline (speedup 1.0000x reference)
import functools
import jax
import jax.numpy as jnp
from jax import lax
from jax.experimental import pallas as pl
from jax.experimental.pallas import tpu as pltpu
from jax.experimental.pallas import tpu_sc as plsc

_B, _N, _D = 4, 4096, 3
_NW = 32
_WPB = _NW // _B
_SL = _N // _WPB
_NCH = _N // 16
_SCH = _SL // 16
_HUGE = 3.0e38


def _direction_pass(cva, cvb, n2b, slice_base, ssum):
    ax, ay, az = cva
    bx, by, bz, b2 = cvb
    lanes = jax.lax.iota(jnp.int32, 16)

    _gdims = lax.GatherDimensionNumbers(
        offset_dims=(), collapsed_slice_dims=(0,), start_index_map=(0,))

    def _bcast(v, k):
        idx = jnp.full((16, 1), k, jnp.int32)
        return lax.gather(v, idx, _gdims, (1,),
                          mode=lax.GatherScatterMode.PROMISE_IN_BOUNDS)

    rot_idx = [jnp.reshape((lanes + r) & 15, (16, 1)) for r in (8, 4, 2, 1)]

    def _hmin_all(v):
        for ri in rot_idx:
            v = jnp.minimum(
                v, lax.gather(v, ri, _gdims, (1,),
                              mode=lax.GatherScatterMode.PROMISE_IN_BOUNDS))
        return v

    _ones = jnp.full((16,), 1.0, jnp.float32)
    _zeros = jnp.zeros((16,), jnp.float32)
    onehots = [jnp.where(lanes == k, _ones, _zeros) for k in range(16)]

    def chunk_body(c, ssum):
        base = slice_base + c * 16
        pxc = ax[pl.ds(base, 16)]
        pyc = ay[pl.ds(base, 16)]
        pzc = az[pl.ds(base, 16)]
        res = jnp.zeros((16,), jnp.float32)
        for half in range(2):
            bxs, bys, bzs = [], [], []
            for k in range(8):
                kk = half * 8 + k
                bxs.append(_bcast(pxc, kk) * (-2.0))
                bys.append(_bcast(pyc, kk) * (-2.0))
                bzs.append(_bcast(pzc, kk) * (-2.0))

            def jbody(j, accs):
                o = j * 16
                gx = bx[pl.ds(o, 16)]
                gy = by[pl.ds(o, 16)]
                gz = bz[pl.ds(o, 16)]
                gg = b2[pl.ds(o, 16)]
                out = []
                for k in range(8):
                    t = gg + bxs[k] * gx
                    t = t + bys[k] * gy
                    t = t + bzs[k] * gz
                    out.append(jnp.minimum(accs[k], t))
                return tuple(out)

            accs = lax.fori_loop(
                0, _NCH, jbody, tuple(jnp.full((16,), _HUGE) for _ in range(8)))
            for k in range(8):
                res = res + _hmin_all(accs[k]) * onehots[half * 8 + k]
        own2 = n2b[pl.ds(base, 16)]
        return ssum + jnp.maximum(res + own2, 0.0)

    return lax.fori_loop(0, _SCH, chunk_body, ssum)


def _sc_body(px_h, py_h, pz_h, gx_h, gy_h, gz_h,
             rpx_h, rpy_h, rpz_h, rgx_h, rgy_h, rgz_h, out_h,
             pxv, pyv, pzv, gxv, gyv, gzv, p2v, g2v, ovec):
    wid = lax.axis_index("s") * 2 + lax.axis_index("c")
    b = wid // _WPB
    slot = wid % _WPB
    bbase = b * _N
    sbase = slot * _SL

    for src, dst in ((px_h, pxv), (py_h, pyv), (pz_h, pzv),
                     (gx_h, gxv), (gy_h, gyv), (gz_h, gzv)):
        pltpu.sync_copy(src.at[pl.ds(bbase, _N)], dst)

    def prep_body(j, _):
        o = j * 16
        for cx, cy, cz, n2 in ((pxv, pyv, pzv, p2v), (gxv, gyv, gzv, g2v)):
            x = cx[pl.ds(o, 16)]
            y = cy[pl.ds(o, 16)]
            z = cz[pl.ds(o, 16)]
            n2[pl.ds(o, 16)] = x * x + y * y + z * z
        return 0

    lax.fori_loop(0, _NCH, prep_body, 0)

    for src, dst in ((rpx_h, pxv), (rpy_h, pyv), (rpz_h, pzv),
                     (rgx_h, gxv), (rgy_h, gyv), (rgz_h, gzv)):
        pltpu.sync_copy(src.at[pl.ds(bbase, _N)], dst)

    ssum = jnp.zeros((16,), jnp.float32)
    ssum = _direction_pass((pxv, pyv, pzv), (gxv, gyv, gzv, g2v), p2v,
                           sbase, ssum)
    ssum = _direction_pass((gxv, gyv, gzv), (pxv, pyv, pzv, p2v), g2v,
                           sbase, ssum)

    ovec[...] = ssum
    pltpu.sync_copy(ovec, out_h.at[wid])


_sc_kernel = functools.partial(
    pl.kernel,
    out_type=jax.ShapeDtypeStruct((_NW, 16), jnp.float32),
    mesh=plsc.VectorSubcoreMesh(core_axis_name="c", subcore_axis_name="s"),
    scratch_types=[pltpu.VMEM((_N,), jnp.float32)] * 8
                  + [pltpu.VMEM((16,), jnp.float32)],
)(_sc_body)


def kernel(pred_pc, gt_pc):
    px = pred_pc[..., 0].reshape(-1)
    py = pred_pc[..., 1].reshape(-1)
    pz = pred_pc[..., 2].reshape(-1)
    gx = gt_pc[..., 0].reshape(-1)
    gy = gt_pc[..., 1].reshape(-1)
    gz = gt_pc[..., 2].reshape(-1)
    def _r(v):
        bits = lax.bitcast_convert_type(v, jnp.int32)
        lsb = lax.shift_right_logical(bits, 16) & 1
        rounded = (bits + 32767 + lsb) & -65536
        return lax.bitcast_convert_type(rounded, jnp.float32)

    part = _sc_kernel(px, py, pz, gx, gy, gz,
                      _r(px), _r(py), _r(pz), _r(gx), _r(gy), _r(gz))
    return jnp.sum(part) * (1.0 / (_B * _N))
